# baseline (speedup 1.0000x reference)
import functools

import jax
import jax.numpy as jnp
from jax import lax
from jax.experimental import pallas as pl
from jax.experimental.pallas import tpu as pltpu
from jax.experimental.pallas import tpu_sc as plsc

NC, NS = 2, 16
NW = NC * NS


def kernel(x, idxs):
    B, C, T, H, W = x.shape
    xt = jnp.transpose(x, (0, 2, 3, 4, 1))
    idxs32 = idxs.astype(jnp.int32)

    SLABS = B * T
    SPW = SLABS // NW
    HCH = 8
    NCHK = H // HCH

    mesh = plsc.VectorSubcoreMesh(core_axis_name="c", subcore_axis_name="s")

    @functools.partial(
        pl.kernel,
        mesh=mesh,
        out_type=jax.ShapeDtypeStruct((B, T, H, W, C), jnp.float32),
        scratch_types=[
            pltpu.VMEM((2, HCH, W, C), jnp.float32),
            pltpu.VMEM((T + 16,), jnp.int32),
            pltpu.SemaphoreType.DMA,
        ],
        compiler_params=pltpu.CompilerParams(use_tc_tiling_on_sc=True),
    )
    def run(x_hbm, idx_hbm, out_hbm, bufs, idx_v, sem):
        wid = lax.axis_index("s") * NC + lax.axis_index("c")

        pltpu.sync_copy(idx_hbm, idx_v.at[pl.ds(0, T)])
        tbase = lax.rem(wid, T // SPW) * SPW
        tvec = idx_v[pl.ds(tbase, 16)]

        for j in range(SPW):
            s = wid * SPW + j
            b = s // T
            tout = tbase + j
            tsrc = tvec[j]

            def in_start(c, slot):
                pltpu.async_copy(
                    x_hbm.at[b, tsrc, pl.ds(c * HCH, HCH)],
                    bufs.at[slot],
                    sem,
                )

            def in_wait(slot):
                pltpu.make_async_copy(
                    x_hbm.at[b, 0, pl.ds(0, HCH)],
                    bufs.at[slot],
                    sem,
                ).wait()

            in_start(0, 0)
            for c in range(NCHK):
                if c + 1 < NCHK:
                    in_start(c + 1, (c + 1) % 2)
                in_wait(c % 2)
                pltpu.sync_copy(
                    bufs.at[c % 2],
                    out_hbm.at[b, tout, pl.ds(c * HCH, HCH)],
                )

    out_t = run(xt, idxs32)
    return jnp.transpose(out_t, (0, 4, 1, 2, 3))


# speedup vs baseline: 1.2004x; 1.2004x over previous
import jax
import jax.numpy as jnp
from jax.experimental import pallas as pl
from jax.experimental.pallas import tpu as pltpu


def _copy_body(idx_ref, x_ref, o_ref):
    o_ref[...] = x_ref[...]


def kernel(x, idxs):
    B, C, T, H, W = x.shape
    xt = jnp.transpose(x, (0, 2, 3, 4, 1))
    idxs32 = idxs.astype(jnp.int32)

    out_t = pl.pallas_call(
        _copy_body,
        grid_spec=pltpu.PrefetchScalarGridSpec(
            num_scalar_prefetch=1,
            grid=(B // 8, T),
            in_specs=[
                pl.BlockSpec(
                    (8, 1, H, W, C),
                    lambda b, t, idx_ref: (b, idx_ref[t], 0, 0, 0),
                )
            ],
            out_specs=pl.BlockSpec(
                (8, 1, H, W, C),
                lambda b, t, idx_ref: (b, t, 0, 0, 0),
            ),
        ),
        out_shape=jax.ShapeDtypeStruct((B, T, H, W, C), x.dtype),
        compiler_params=pltpu.CompilerParams(vmem_limit_bytes=120 * 1024 * 1024),
    )(idxs32, xt)
    return jnp.transpose(out_t, (0, 4, 1, 2, 3))
